# baseline (speedup 1.0000x reference)
import jax
import jax.numpy as jnp
import numpy as np
from jax.experimental import pallas as pl
from jax.experimental.pallas import tpu as pltpu

_B = 1024
_P = 64

_SPACE = np.array([8000.0, 8000.0, 2000.0], np.float32)
_VOX = np.array([80.0, 80.0, 20.0], np.float32)
_CENTER = np.array([0.0, 0.0, 1000.0], np.float32)
_SCALE = _SPACE / (_VOX - 1.0)
_BIAS = _CENTER - _SPACE / 2.0
_MIN_SCORE = 0.3

_NC = 8
_R = _P // _NC


def _body(idx_hbm, conf_hbm, bbox_hbm, out_hbm, idx_v, conf_v, bbox_v, out_v, sin, sout):
    sx, sy, sz = float(_SCALE[0]), float(_SCALE[1]), float(_SCALE[2])
    bx, by, bz = float(_BIAS[0]), float(_BIAS[1]), float(_BIAS[2])

    def in_copies(c):
        sl = pl.ds(_R * c, _R)
        return [
            pltpu.make_async_copy(idx_hbm.at[:, sl, :], idx_v.at[:, sl, :], sin.at[c]),
            pltpu.make_async_copy(conf_hbm.at[sl, :], conf_v.at[sl, :], sin.at[c]),
            pltpu.make_async_copy(bbox_hbm.at[sl, :, :], bbox_v.at[sl, :, :], sin.at[c]),
        ]

    def out_copy(c):
        sl = pl.ds(_R * c, _R)
        return pltpu.make_async_copy(out_v.at[:, sl, :], out_hbm.at[:, sl, :], sout.at[c])

    for cp in in_copies(0):
        cp.start()
    for cp in in_copies(1):
        cp.start()
    for c in range(_NC):
        for cp in in_copies(c):
            cp.wait()
        sl = pl.ds(_R * c, _R)
        idxf = idx_v[:, sl, :].astype(jnp.float32)
        out_v[0, sl, :] = idxf[0] * sx + bx
        out_v[1, sl, :] = idxf[1] * sy + by
        out_v[2, sl, :] = idxf[2] * sz + bz
        cf = conf_v[sl, :]
        out_v[3, sl, :] = (cf > _MIN_SCORE).astype(jnp.float32) - 1.0
        out_v[4, sl, :] = cf
        out_v[5, sl, :] = bbox_v[sl, 0, :]
        out_v[6, sl, :] = bbox_v[sl, 1, :]
        out_copy(c).start()
        if c + 2 < _NC:
            for cp in in_copies(c + 2):
                cp.start()
    for c in range(_NC):
        out_copy(c).wait()


@jax.jit
def _proposal_tc(idx_t, conf_t, bbox_t):
    any_spec = pl.BlockSpec(memory_space=pltpu.MemorySpace.HBM)
    return pl.pallas_call(
        _body,
        in_specs=[any_spec, any_spec, any_spec],
        out_specs=any_spec,
        out_shape=jax.ShapeDtypeStruct((7, _P, _B), jnp.float32),
        scratch_shapes=[
            pltpu.VMEM((3, _P, _B), jnp.int32),
            pltpu.VMEM((_P, _B), jnp.float32),
            pltpu.VMEM((_P, 2, _B), jnp.float32),
            pltpu.VMEM((7, _P, _B), jnp.float32),
            pltpu.SemaphoreType.DMA((_NC,)),
            pltpu.SemaphoreType.DMA((_NC,)),
        ],
    )(idx_t, conf_t, bbox_t)


def kernel(topk_index, topk_confs, match_bbox_preds, meta):
    del meta
    idx_t = jnp.transpose(topk_index, (2, 1, 0))
    conf_t = jnp.transpose(topk_confs, (1, 0))
    bbox_t = jnp.transpose(match_bbox_preds, (1, 2, 0))
    out_t = _proposal_tc(idx_t, conf_t, bbox_t)
    return jnp.transpose(out_t, (2, 1, 0))


# speedup vs baseline: 1.9606x; 1.9606x over previous
import jax
import jax.numpy as jnp
import numpy as np
from jax.experimental import pallas as pl
from jax.experimental.pallas import tpu as pltpu

_B = 1024
_P = 64

_SPACE = np.array([8000.0, 8000.0, 2000.0], np.float32)
_VOX = np.array([80.0, 80.0, 20.0], np.float32)
_CENTER = np.array([0.0, 0.0, 1000.0], np.float32)
_SCALE = _SPACE / (_VOX - 1.0)
_BIAS = _CENTER - _SPACE / 2.0
_MIN_SCORE = 0.3

_NC = 2
_R = _P // _NC


def _body(idx_hbm, conf_hbm, bbox_hbm, out_hbm, idx_v, conf_v, bbox_v, out_v, sin, sout):
    sx, sy, sz = float(_SCALE[0]), float(_SCALE[1]), float(_SCALE[2])
    bx, by, bz = float(_BIAS[0]), float(_BIAS[1]), float(_BIAS[2])

    def in_copies(c):
        sl = pl.ds(_R * c, _R)
        return [
            pltpu.make_async_copy(idx_hbm.at[:, sl, :], idx_v.at[:, sl, :], sin.at[c]),
            pltpu.make_async_copy(conf_hbm.at[sl, :], conf_v.at[sl, :], sin.at[c]),
            pltpu.make_async_copy(bbox_hbm.at[sl, :, :], bbox_v.at[sl, :, :], sin.at[c]),
        ]

    def out_copy(c):
        sl = pl.ds(_R * c, _R)
        return pltpu.make_async_copy(out_v.at[:, sl, :], out_hbm.at[:, sl, :], sout.at[c])

    for cp in in_copies(0):
        cp.start()
    for cp in in_copies(1):
        cp.start()
    for c in range(_NC):
        for cp in in_copies(c):
            cp.wait()
        sl = pl.ds(_R * c, _R)
        idxf = idx_v[:, sl, :].astype(jnp.float32)
        out_v[0, sl, :] = idxf[0] * sx + bx
        out_v[1, sl, :] = idxf[1] * sy + by
        out_v[2, sl, :] = idxf[2] * sz + bz
        cf = conf_v[sl, :]
        out_v[3, sl, :] = (cf > _MIN_SCORE).astype(jnp.float32) - 1.0
        out_v[4, sl, :] = cf
        out_v[5, sl, :] = bbox_v[sl, 0, :]
        out_v[6, sl, :] = bbox_v[sl, 1, :]
        out_copy(c).start()
        if c + 2 < _NC:
            for cp in in_copies(c + 2):
                cp.start()
    for c in range(_NC):
        out_copy(c).wait()


@jax.jit
def _proposal_tc(idx_t, conf_t, bbox_t):
    any_spec = pl.BlockSpec(memory_space=pltpu.MemorySpace.HBM)
    return pl.pallas_call(
        _body,
        in_specs=[any_spec, any_spec, any_spec],
        out_specs=any_spec,
        out_shape=jax.ShapeDtypeStruct((7, _P, _B), jnp.float32),
        scratch_shapes=[
            pltpu.VMEM((3, _P, _B), jnp.int32),
            pltpu.VMEM((_P, _B), jnp.float32),
            pltpu.VMEM((_P, 2, _B), jnp.float32),
            pltpu.VMEM((7, _P, _B), jnp.float32),
            pltpu.SemaphoreType.DMA((_NC,)),
            pltpu.SemaphoreType.DMA((_NC,)),
        ],
    )(idx_t, conf_t, bbox_t)


def kernel(topk_index, topk_confs, match_bbox_preds, meta):
    del meta
    idx_t = jnp.transpose(topk_index, (2, 1, 0))
    conf_t = jnp.transpose(topk_confs, (1, 0))
    bbox_t = jnp.transpose(match_bbox_preds, (1, 2, 0))
    out_t = _proposal_tc(idx_t, conf_t, bbox_t)
    return jnp.transpose(out_t, (2, 1, 0))
